# baseline (speedup 1.0000x reference)
import functools

import jax
import jax.numpy as jnp
from jax import lax
from jax.experimental import pallas as pl
from jax.experimental.pallas import tpu as pltpu


def _round_up(a, m):
    return (a + m - 1) // m * m


def _prep_kernel(eps2, inv_temp, x_ref, y_ref, xn_ref, yn_ref):
    xf = x_ref[...]
    yf = y_ref[...]
    inv_nx = lax.rsqrt(
        jnp.maximum(jnp.sum(xf * xf, axis=1, keepdims=True), eps2)) * inv_temp
    inv_ny = lax.rsqrt(
        jnp.maximum(jnp.sum(yf * yf, axis=1, keepdims=True), eps2))
    xn_ref[...] = (xf * inv_nx).astype(jnp.bfloat16)
    yn_ref[...] = (yf * inv_ny).astype(jnp.bfloat16)


def _prep_one_kernel(eps2, scale, x_ref, xn_ref):
    xf = x_ref[...]
    inv_n = lax.rsqrt(
        jnp.maximum(jnp.sum(xf * xf, axis=1, keepdims=True), eps2)) * scale
    xn_ref[...] = (xf * inv_n).astype(jnp.bfloat16)


def _matmul_kernel(xn_ref, yn_ref, o_ref):
    o_ref[...] = lax.dot_general(
        xn_ref[...], yn_ref[...],
        dimension_numbers=(((1,), (1,)), ((), ())),
        preferred_element_type=jnp.float32)


def _fused_kernel(eps2, inv_temp, tm, nsteps,
                  x_hbm, y_ref, o_ref, yn_ref, xs_ref, sem_ref):
    i = pl.program_id(0)

    def _copy(step, slot):
        return pltpu.make_async_copy(
            x_hbm.at[pl.ds(step * tm, tm), 0, :],
            xs_ref.at[slot],
            sem_ref.at[slot],
        )

    @pl.when(i == 0)
    def _():
        _copy(0, 0).start()
        yf = y_ref[0]
        inv_ny = lax.rsqrt(
            jnp.maximum(jnp.sum(yf * yf, axis=1, keepdims=True), eps2))
        yn_ref[...] = (yf * inv_ny).astype(jnp.bfloat16)

    @pl.when(i + 1 < nsteps)
    def _():
        _copy(i + 1, (i + 1) % 2).start()

    slot = i % 2
    _copy(i, slot).wait()
    xf = xs_ref[slot]
    inv_nx = lax.rsqrt(
        jnp.maximum(jnp.sum(xf * xf, axis=1, keepdims=True), eps2)) * inv_temp
    xn = (xf * inv_nx).astype(jnp.bfloat16)
    o_ref[...] = lax.dot_general(
        xn, yn_ref[...],
        dimension_numbers=(((1,), (1,)), ((), ())),
        preferred_element_type=jnp.float32)


def _normalize_rows(rows, eps2, scale, tile):
    b, h = rows.shape
    grid = (b // tile,)
    return pl.pallas_call(
        functools.partial(_prep_one_kernel, eps2, scale),
        out_shape=jax.ShapeDtypeStruct((b, h), jnp.bfloat16),
        grid=grid,
        in_specs=[pl.BlockSpec((tile, h), lambda i: (i, 0))],
        out_specs=pl.BlockSpec((tile, h), lambda i: (i, 0)),
        compiler_params=pltpu.CompilerParams(
            dimension_semantics=("parallel",)),
    )(rows)


def kernel(x, y, temp=0.05, eps=1e-8):
    inv_temp_ = 1.0 / float(temp)
    eps2_ = float(eps) ** 2

    if (x.ndim == 3 and y.ndim == 3 and x.shape[1] == 1 and y.shape[0] == 1
            and x.dtype == jnp.float32 and y.dtype == jnp.float32):
        b1, _, h = x.shape
        b2 = y.shape[1]
        tm = 512
        fused_need = (b2 * h * 6 + 2 * tm * h * 4 + 2 * tm * b2 * 4
                      + b2 * h * 2)
        if (b1 % tm == 0 and b2 % 128 == 0 and h % 128 == 0
                and fused_need <= 58 * 1024 * 1024):
            nsteps = b1 // tm
            grid = (nsteps,)
            cost = pl.CostEstimate(
                flops=2 * b1 * b2 * h,
                transcendentals=b1 + b2,
                bytes_accessed=(b1 * h * 4 + b2 * h * 4 + b1 * b2 * 4),
            )
            return pl.pallas_call(
                functools.partial(_fused_kernel, eps2_, inv_temp_, tm, nsteps),
                out_shape=jax.ShapeDtypeStruct((b1, b2), jnp.float32),
                grid=grid,
                in_specs=[
                    pl.BlockSpec(memory_space=pl.ANY),
                    pl.BlockSpec((1, b2, h), lambda i: (0, 0, 0)),
                ],
                out_specs=pl.BlockSpec((tm, b2), lambda i: (i, 0)),
                scratch_shapes=[
                    pltpu.VMEM((b2, h), jnp.bfloat16),
                    pltpu.VMEM((2, tm, h), jnp.float32),
                    pltpu.SemaphoreType.DMA((2,)),
                ],
                compiler_params=pltpu.CompilerParams(
                    dimension_semantics=("arbitrary",),
                    vmem_limit_bytes=58 * 1024 * 1024,
                ),
                cost_estimate=cost,
            )(x, y)

    if x.ndim == 3:
        x_rows = x[:, 0, :]
    else:
        x_rows = x
    if y.ndim == 3:
        y_rows = y[0, :, :]
    else:
        y_rows = y

    b1, h = x_rows.shape
    b2, _ = y_rows.shape

    inv_temp = 1.0 / float(temp)
    eps2 = float(eps) ** 2

    tm = min(512, _round_up(b1, 8))
    b1p = _round_up(b1, tm)
    b2p = _round_up(b2, 128)
    if b1p != b1:
        x_rows = jnp.pad(x_rows, ((0, b1p - b1), (0, 0)))
    if b2p != b2:
        y_rows = jnp.pad(y_rows, ((0, b2p - b2), (0, 0)))

    if b1p == b2p:
        tp = min(512, b1p)
        grid_p = (b1p // tp,)
        xn, yn = pl.pallas_call(
            functools.partial(_prep_kernel, eps2, inv_temp),
            out_shape=(
                jax.ShapeDtypeStruct((b1p, h), jnp.bfloat16),
                jax.ShapeDtypeStruct((b2p, h), jnp.bfloat16),
            ),
            grid=grid_p,
            in_specs=[
                pl.BlockSpec((tp, h), lambda i: (i, 0)),
                pl.BlockSpec((tp, h), lambda i: (i, 0)),
            ],
            out_specs=(
                pl.BlockSpec((tp, h), lambda i: (i, 0)),
                pl.BlockSpec((tp, h), lambda i: (i, 0)),
            ),
            compiler_params=pltpu.CompilerParams(
                dimension_semantics=("parallel",)),
        )(x_rows, y_rows)
    else:
        xn = _normalize_rows(x_rows, eps2, inv_temp, min(512, b1p))
        yn = _normalize_rows(y_rows, eps2, 1.0, min(512, b2p))

    grid = (b1p // tm,)
    cost = pl.CostEstimate(
        flops=2 * b1p * b2p * h,
        transcendentals=0,
        bytes_accessed=(b1p * h * 2 + b2p * h * 2 + b1p * b2p * 4),
    )
    out = pl.pallas_call(
        _matmul_kernel,
        out_shape=jax.ShapeDtypeStruct((b1p, b2p), jnp.float32),
        grid=grid,
        in_specs=[
            pl.BlockSpec((tm, h), lambda i: (i, 0)),
            pl.BlockSpec((b2p, h), lambda i: (0, 0)),
        ],
        out_specs=pl.BlockSpec((tm, b2p), lambda i: (i, 0)),
        compiler_params=pltpu.CompilerParams(
            dimension_semantics=("parallel",),
            vmem_limit_bytes=52 * 1024 * 1024,
        ),
        cost_estimate=cost,
    )(xn, yn)

    if (b1p, b2p) != (b1, b2):
        out = out[:b1, :b2]
    return out


# speedup vs baseline: 1.0159x; 1.0159x over previous
import functools

import jax
import jax.numpy as jnp
from jax import lax
from jax.experimental import pallas as pl
from jax.experimental.pallas import tpu as pltpu


def _round_up(a, m):
    return (a + m - 1) // m * m


def _prep_kernel(eps2, inv_temp, x_ref, y_ref, xn_ref, yn_ref):
    xf = x_ref[...]
    yf = y_ref[...]
    inv_nx = lax.rsqrt(
        jnp.maximum(jnp.sum(xf * xf, axis=1, keepdims=True), eps2)) * inv_temp
    inv_ny = lax.rsqrt(
        jnp.maximum(jnp.sum(yf * yf, axis=1, keepdims=True), eps2))
    xn_ref[...] = (xf * inv_nx).astype(jnp.bfloat16)
    yn_ref[...] = (yf * inv_ny).astype(jnp.bfloat16)


def _prep_one_kernel(eps2, scale, x_ref, xn_ref):
    xf = x_ref[...]
    inv_n = lax.rsqrt(
        jnp.maximum(jnp.sum(xf * xf, axis=1, keepdims=True), eps2)) * scale
    xn_ref[...] = (xf * inv_n).astype(jnp.bfloat16)


def _matmul_kernel(xn_ref, yn_ref, o_ref):
    o_ref[...] = lax.dot_general(
        xn_ref[...], yn_ref[...],
        dimension_numbers=(((1,), (1,)), ((), ())),
        preferred_element_type=jnp.float32)


def _fused_kernel(eps2, inv_temp, tm, nsteps, ycs,
                  x_hbm, y_hbm, o_ref, yn_ref, xs_ref, ys_ref,
                  sem_ref, ysem_ref):
    i = pl.program_id(0)

    def _copy(step, slot):
        return pltpu.make_async_copy(
            x_hbm.at[pl.ds(step * tm, tm), 0, :],
            xs_ref.at[slot],
            sem_ref.at[slot],
        )

    nyc = ys_ref.shape[0] // ycs

    def _ycopy(c):
        return pltpu.make_async_copy(
            y_hbm.at[0, pl.ds(c * ycs, ycs), :],
            ys_ref.at[pl.ds(c * ycs, ycs), :],
            ysem_ref.at[c],
        )

    @pl.when(i == 0)
    def _():
        _copy(0, 0).start()
        for c in range(nyc):
            _ycopy(c).start()
        for c in range(nyc):
            _ycopy(c).wait()
            yf = ys_ref[pl.ds(c * ycs, ycs), :]
            inv_ny = lax.rsqrt(
                jnp.maximum(jnp.sum(yf * yf, axis=1, keepdims=True), eps2))
            yn_ref[pl.ds(c * ycs, ycs), :] = (yf * inv_ny).astype(jnp.bfloat16)

    @pl.when(i + 1 < nsteps)
    def _():
        _copy(i + 1, (i + 1) % 2).start()

    slot = i % 2
    _copy(i, slot).wait()
    xf = xs_ref[slot]
    inv_nx = lax.rsqrt(
        jnp.maximum(jnp.sum(xf * xf, axis=1, keepdims=True), eps2)) * inv_temp
    xn = (xf * inv_nx).astype(jnp.bfloat16)
    o_ref[...] = lax.dot_general(
        xn, yn_ref[...],
        dimension_numbers=(((1,), (1,)), ((), ())),
        preferred_element_type=jnp.float32)


def _normalize_rows(rows, eps2, scale, tile):
    b, h = rows.shape
    grid = (b // tile,)
    return pl.pallas_call(
        functools.partial(_prep_one_kernel, eps2, scale),
        out_shape=jax.ShapeDtypeStruct((b, h), jnp.bfloat16),
        grid=grid,
        in_specs=[pl.BlockSpec((tile, h), lambda i: (i, 0))],
        out_specs=pl.BlockSpec((tile, h), lambda i: (i, 0)),
        compiler_params=pltpu.CompilerParams(
            dimension_semantics=("parallel",)),
    )(rows)


def kernel(x, y, temp=0.05, eps=1e-8):
    inv_temp_ = 1.0 / float(temp)
    eps2_ = float(eps) ** 2

    if (x.ndim == 3 and y.ndim == 3 and x.shape[1] == 1 and y.shape[0] == 1
            and x.dtype == jnp.float32 and y.dtype == jnp.float32):
        b1, _, h = x.shape
        b2 = y.shape[1]
        tm = 512
        fused_need = (b2 * h * 6 + 2 * tm * h * 4 + 2 * tm * b2 * 4
                      + b2 * h * 2)
        if (b1 % tm == 0 and b2 % 128 == 0 and h % 128 == 0
                and fused_need <= 58 * 1024 * 1024):
            nsteps = b1 // tm
            ycs = b2 // 4 if b2 % 4 == 0 else b2
            nyc = b2 // ycs
            grid = (nsteps,)
            cost = pl.CostEstimate(
                flops=2 * b1 * b2 * h,
                transcendentals=b1 + b2,
                bytes_accessed=(b1 * h * 4 + b2 * h * 4 + b1 * b2 * 4),
            )
            return pl.pallas_call(
                functools.partial(_fused_kernel, eps2_, inv_temp_, tm,
                                  nsteps, ycs),
                out_shape=jax.ShapeDtypeStruct((b1, b2), jnp.float32),
                grid=grid,
                in_specs=[
                    pl.BlockSpec(memory_space=pl.ANY),
                    pl.BlockSpec(memory_space=pl.ANY),
                ],
                out_specs=pl.BlockSpec((tm, b2), lambda i: (i, 0)),
                scratch_shapes=[
                    pltpu.VMEM((b2, h), jnp.bfloat16),
                    pltpu.VMEM((2, tm, h), jnp.float32),
                    pltpu.VMEM((b2, h), jnp.float32),
                    pltpu.SemaphoreType.DMA((2,)),
                    pltpu.SemaphoreType.DMA((nyc,)),
                ],
                compiler_params=pltpu.CompilerParams(
                    dimension_semantics=("arbitrary",),
                    vmem_limit_bytes=58 * 1024 * 1024,
                ),
                cost_estimate=cost,
            )(x, y)

    if x.ndim == 3:
        x_rows = x[:, 0, :]
    else:
        x_rows = x
    if y.ndim == 3:
        y_rows = y[0, :, :]
    else:
        y_rows = y

    b1, h = x_rows.shape
    b2, _ = y_rows.shape

    inv_temp = 1.0 / float(temp)
    eps2 = float(eps) ** 2

    tm = min(512, _round_up(b1, 8))
    b1p = _round_up(b1, tm)
    b2p = _round_up(b2, 128)
    if b1p != b1:
        x_rows = jnp.pad(x_rows, ((0, b1p - b1), (0, 0)))
    if b2p != b2:
        y_rows = jnp.pad(y_rows, ((0, b2p - b2), (0, 0)))

    if b1p == b2p:
        tp = min(512, b1p)
        grid_p = (b1p // tp,)
        xn, yn = pl.pallas_call(
            functools.partial(_prep_kernel, eps2, inv_temp),
            out_shape=(
                jax.ShapeDtypeStruct((b1p, h), jnp.bfloat16),
                jax.ShapeDtypeStruct((b2p, h), jnp.bfloat16),
            ),
            grid=grid_p,
            in_specs=[
                pl.BlockSpec((tp, h), lambda i: (i, 0)),
                pl.BlockSpec((tp, h), lambda i: (i, 0)),
            ],
            out_specs=(
                pl.BlockSpec((tp, h), lambda i: (i, 0)),
                pl.BlockSpec((tp, h), lambda i: (i, 0)),
            ),
            compiler_params=pltpu.CompilerParams(
                dimension_semantics=("parallel",)),
        )(x_rows, y_rows)
    else:
        xn = _normalize_rows(x_rows, eps2, inv_temp, min(512, b1p))
        yn = _normalize_rows(y_rows, eps2, 1.0, min(512, b2p))

    grid = (b1p // tm,)
    cost = pl.CostEstimate(
        flops=2 * b1p * b2p * h,
        transcendentals=0,
        bytes_accessed=(b1p * h * 2 + b2p * h * 2 + b1p * b2p * 4),
    )
    out = pl.pallas_call(
        _matmul_kernel,
        out_shape=jax.ShapeDtypeStruct((b1p, b2p), jnp.float32),
        grid=grid,
        in_specs=[
            pl.BlockSpec((tm, h), lambda i: (i, 0)),
            pl.BlockSpec((b2p, h), lambda i: (0, 0)),
        ],
        out_specs=pl.BlockSpec((tm, b2p), lambda i: (i, 0)),
        compiler_params=pltpu.CompilerParams(
            dimension_semantics=("parallel",),
            vmem_limit_bytes=52 * 1024 * 1024,
        ),
        cost_estimate=cost,
    )(xn, yn)

    if (b1p, b2p) != (b1, b2):
        out = out[:b1, :b2]
    return out
